# baseline (speedup 1.0000x reference)
import jax
import jax.numpy as jnp
from jax import lax
from jax.experimental import pallas as pl
from jax.experimental.pallas import tpu as pltpu
from jax.experimental.pallas import tpu_sc as plsc

BATCH = 4096
SEQ = 200
DIM = 64
N = BATCH * SEQ

NUM_CORES = 2
NUM_SUBCORES = 16
NW = NUM_CORES * NUM_SUBCORES
ROWS_PER_W = N // NW

GB = 64
KSUB = 8
CHUNK = GB * KSUB
G = ROWS_PER_W // CHUNK

NBUF = 3

TMPL = (SEQ + CHUNK + 7) // 8 * 8


def _body(xf_hbm, table_hbm, tmpl_hbm, out_hbm,
          iv0, iv1, iv2, b0, b1, b2,
          si0, si1, si2, sg0, sg1, sg2, ss0, ss1, ss2):
    idx_v = [iv0, iv1, iv2]
    buf = [b0, b1, b2]
    sem_it = [si0, si1, si2]
    sem_g = [sg0, sg1, sg2]
    sem_s = [ss0, ss1, ss2]

    wid = lax.axis_index("s") * NUM_CORES + lax.axis_index("c")
    base0 = wid * ROWS_PER_W

    def in_copies(g, b):
        idx_off = pl.multiple_of(base0 // GB + g * KSUB, KSUB)
        tmpl_off = pl.multiple_of(lax.rem(g * CHUNK, SEQ), 8)
        return (
            pltpu.make_async_copy(xf_hbm.at[pl.ds(idx_off, KSUB)], idx_v[b], sem_it[b]),
            pltpu.make_async_copy(tmpl_hbm.at[pl.ds(tmpl_off, CHUNK)], buf[b], sem_it[b]),
        )

    def gather_copies(b):
        return [
            pltpu.make_async_copy(
                table_hbm.at[idx_v[b].at[k]],
                buf[b].at[pl.ds(k * GB, GB)],
                sem_g[b],
            )
            for k in range(KSUB)
        ]

    def out_copy(g, b):
        base = pl.multiple_of(base0 + g * CHUNK, CHUNK)
        return pltpu.make_async_copy(buf[b], out_hbm.at[pl.ds(base, CHUNK)], sem_s[b])

    def step(g, b, bp, bn, first, last):
        for c in in_copies(g, b):
            c.wait()
        gs = gather_copies(b)
        for c in gs:
            c.start(add=True)
        if not first:
            for c in gather_copies(bp):
                c.wait()
            out_copy(g - 1, bp).start()
        if not last:
            def prefetch(gn):
                if gn >= NBUF:
                    out_copy(gn - NBUF, bn).wait()
                for c in in_copies(gn, bn):
                    c.start()
            if isinstance(g, int):
                prefetch(g + 1)
            else:
                out_copy(g + 1 - NBUF, bn).wait()
                for c in in_copies(g + 1, bn):
                    c.start()

    for c in in_copies(0, 0):
        c.start()
    step(0, 0, 2, 1, True, False)
    step(1, 1, 0, 2, False, False)

    RB = (G - 3) // NBUF
    def ring(blk, carry):
        g0 = blk * NBUF
        step(g0 + 2, 2, 1, 0, False, False)
        step(g0 + 3, 0, 2, 1, False, False)
        step(g0 + 4, 1, 0, 2, False, False)
        return carry

    lax.fori_loop(0, RB, ring, 0)

    for g in range(2 + NBUF * RB, G):
        b, bp, bn = g % NBUF, (g - 1) % NBUF, (g + 1) % NBUF
        step(g, b, bp, bn, False, g == G - 1)
    gl = G - 1
    for c in gather_copies(gl % NBUF):
        c.wait()
    out_copy(gl, gl % NBUF).start()
    for g in range(gl - 2, gl + 1):
        out_copy(g, g % NBUF).wait()


@jax.jit
def _run(xf, table, tmpl):
    mesh = plsc.VectorSubcoreMesh(core_axis_name="c", subcore_axis_name="s")
    f = pl.kernel(
        _body,
        out_type=jax.ShapeDtypeStruct((N, DIM), jnp.float32),
        mesh=mesh,
        compiler_params=pltpu.CompilerParams(use_tc_tiling_on_sc=False),
        scratch_types=(
            [pltpu.VMEM((KSUB, GB), jnp.int32) for _ in range(NBUF)]
            + [pltpu.VMEM((CHUNK, DIM), jnp.float32) for _ in range(NBUF)]
            + [pltpu.SemaphoreType.DMA for _ in range(3 * NBUF)]
        ),
    )
    return f(xf, table, tmpl)


def kernel(x, table, pos_encoding):
    xf = x.reshape(N // GB, GB).astype(jnp.int32)
    reps = -(-TMPL // SEQ)
    tmpl = jnp.tile(pos_encoding[:SEQ], (reps, 1))[:TMPL]
    out = _run(xf, table, tmpl)
    return out.reshape(BATCH, SEQ, DIM)
